# Initial kernel scaffold; baseline (speedup 1.0000x reference)
import jax
import jax.numpy as jnp
from jax.experimental import pallas as pl


def kernel(x, edge_index, batch, W1a, b1a, W1b, b1b, W2a, b2a, W2b, b2b, W3a, b3a, W3b, b3b, Wc1, bc1, Wc2, bc2):
    raise NotImplementedError("write your pallas kernel here")


# speedup vs baseline: 4.0183x; 4.0183x over previous
import functools

import jax
import jax.numpy as jnp
from jax import lax
from jax.experimental import pallas as pl
from jax.experimental.pallas import tpu as pltpu
from jax.experimental.pallas import tpu_sc as plsc

_NC = 2
_NS = 16
_C = 128


def _sc_gather_scatter_add(x, src_p, dst_p, n_pad):
    nw, K, C = src_p.shape
    N, D = x.shape
    rows_init = (N // (_NS * 8)) * 8
    init_tail = N - _NS * rows_init
    rows_out = n_pad // _NS
    mesh = plsc.VectorSubcoreMesh(core_axis_name="c", subcore_axis_name="s")

    @functools.partial(
        pl.kernel,
        mesh=mesh,
        out_type=jax.ShapeDtypeStruct((_NC, n_pad, D), jnp.float32),
        scratch_types=[
            pltpu.VMEM((K, C), jnp.int32),
            pltpu.VMEM((K, C), jnp.int32),
            pltpu.VMEM((C, D), jnp.float32),
            pltpu.VMEM_SHARED((n_pad, D), jnp.float32),
        ],
    )
    def k(x_hbm, src_hbm, dst_hbm, out_hbm, srcv, dstv, rows, acc):
        c = lax.axis_index("c")
        s = lax.axis_index("s")
        wid = c * _NS + s
        pltpu.sync_copy(src_hbm.at[wid], srcv)
        pltpu.sync_copy(dst_hbm.at[wid], dstv)
        pltpu.sync_copy(x_hbm.at[pl.ds(s * rows_init, rows_init)],
                        acc.at[pl.ds(s * rows_init, rows_init)])
        if init_tail:
            @pl.when(s == 0)
            def _():
                pltpu.sync_copy(
                    x_hbm.at[pl.ds(_NS * rows_init, init_tail)],
                    acc.at[pl.ds(_NS * rows_init, init_tail)])
        plsc.subcore_barrier()

        @pl.loop(0, K)
        def _(j):
            pltpu.sync_copy(x_hbm.at[srcv.at[j]], rows)
            pltpu.sync_copy(rows, acc.at[dstv.at[j]], add=True)

        plsc.subcore_barrier()
        pltpu.sync_copy(acc.at[pl.ds(s * rows_out, rows_out)],
                        out_hbm.at[c, pl.ds(s * rows_out, rows_out)])

    return k(x, src_p, dst_p)


def _mlp(agg, x, Wa, ba, Wb, bb, block_rows):
    N, D = x.shape
    grid = N // block_rows

    def body(agg_ref, x_ref, wa, ba_r, wb, bb_r, o_ref):
        g = agg_ref[0] + agg_ref[1] - x_ref[...]
        h1 = jnp.maximum(
            jnp.dot(g, wa[...], preferred_element_type=jnp.float32) + ba_r[...],
            0.0)
        o_ref[...] = (jnp.dot(h1, wb[...], preferred_element_type=jnp.float32)
                      + bb_r[...])

    return pl.pallas_call(
        body,
        grid=(grid,),
        in_specs=[
            pl.BlockSpec((_NC, block_rows, D), lambda i: (0, i, 0)),
            pl.BlockSpec((block_rows, D), lambda i: (i, 0)),
            pl.BlockSpec((D, D), lambda i: (0, 0)),
            pl.BlockSpec((1, D), lambda i: (0, 0)),
            pl.BlockSpec((D, D), lambda i: (0, 0)),
            pl.BlockSpec((1, D), lambda i: (0, 0)),
        ],
        out_specs=pl.BlockSpec((block_rows, D), lambda i: (i, 0)),
        out_shape=jax.ShapeDtypeStruct((N, D), jnp.float32),
    )(agg, x, Wa, ba.reshape(1, D), Wb, bb.reshape(1, D))


def _pool_classify(h, batch3, Wc1, bc1, Wc2, bc2, num_graphs, block_rows):
    N, D = h.shape
    n_classes = Wc2.shape[1]
    grid = N // block_rows

    def body(h_ref, b_ref, wc1, bc1_r, wc2, bc2_r, o_ref, acc_ref):
        i = pl.program_id(0)

        @pl.when(i == 0)
        def _():
            acc_ref[...] = jnp.zeros_like(acc_ref)

        b = b_ref[0, 0, :]
        onehot = (b[:, None] == lax.broadcasted_iota(
            jnp.int32, (block_rows, num_graphs), 1)).astype(jnp.float32)
        acc_ref[...] += lax.dot_general(
            onehot, h_ref[...], (((0,), (0,)), ((), ())),
            preferred_element_type=jnp.float32)

        @pl.when(i == grid - 1)
        def _():
            t = jnp.maximum(
                jnp.dot(acc_ref[...], wc1[...],
                        preferred_element_type=jnp.float32) + bc1_r[...], 0.0)
            o_ref[...] = (jnp.dot(t, wc2[...],
                                  preferred_element_type=jnp.float32)
                          + bc2_r[...])

    return pl.pallas_call(
        body,
        grid=(grid,),
        in_specs=[
            pl.BlockSpec((block_rows, D), lambda i: (i, 0)),
            pl.BlockSpec((1, 1, block_rows), lambda i: (i, 0, 0)),
            pl.BlockSpec((D, D), lambda i: (0, 0)),
            pl.BlockSpec((1, D), lambda i: (0, 0)),
            pl.BlockSpec((D, n_classes), lambda i: (0, 0)),
            pl.BlockSpec((1, n_classes), lambda i: (0, 0)),
        ],
        out_specs=pl.BlockSpec((num_graphs, n_classes), lambda i: (0, 0)),
        out_shape=jax.ShapeDtypeStruct((num_graphs, n_classes), jnp.float32),
        scratch_shapes=[pltpu.VMEM((num_graphs, D), jnp.float32)],
    )(h, batch3, Wc1, bc1.reshape(1, D), Wc2, bc2.reshape(1, n_classes))


def kernel(x, edge_index, batch, W1a, b1a, W1b, b1b, W2a, b2a, W2b, b2b,
           W3a, b3a, W3b, b3b, Wc1, bc1, Wc2, bc2):
    N, D = x.shape
    E = edge_index.shape[1]
    num_graphs = 64
    nw = _NC * _NS
    K = -(-E // (nw * _C))
    e_pad = nw * K * _C
    n_pad = -(-(N + 1) // (_NS * 8)) * _NS * 8

    src = edge_index[0]
    dst = edge_index[1]
    src_p = jnp.concatenate(
        [src, jnp.zeros((e_pad - E,), jnp.int32)]).reshape(nw, K, _C)
    dst_p = jnp.concatenate(
        [dst, jnp.full((e_pad - E,), N, jnp.int32)]).reshape(nw, K, _C)

    block_rows = 400
    batch3 = batch.reshape(N // block_rows, 1, block_rows)

    h = x
    for (Wa, ba, Wb, bb) in ((W1a, b1a, W1b, b1b), (W2a, b2a, W2b, b2b),
                             (W3a, b3a, W3b, b3b)):
        agg = _sc_gather_scatter_add(h, src_p, dst_p, n_pad)
        h = _mlp(agg, h, Wa, ba, Wb, bb, block_rows)

    return _pool_classify(h, batch3, Wc1, bc1, Wc2, bc2, num_graphs,
                          block_rows)
